# Initial kernel scaffold; baseline (speedup 1.0000x reference)
import jax
import jax.numpy as jnp
from jax.experimental import pallas as pl


def kernel(edge_index, edge_type, head_idx, tail_idx, rel_idx, node_embed, rel_embed, W1, Wl1, Wr1, W2, Wl2, Wr2):
    raise NotImplementedError("write your pallas kernel here")


# speedup vs baseline: 3.9274x; 3.9274x over previous
import functools

import jax
import jax.numpy as jnp
from jax import lax
from jax.experimental import pallas as pl
from jax.experimental.pallas import tpu as pltpu
from jax.experimental.pallas import tpu_sc as plsc

N_NODES = 10000
N_REL = 500
N_EDGES = 320000
DIM = 128
BATCH = 16384

NC = 2
NS = 16
NW = NC * NS

EDGES_PER_W = N_EDGES // NW
K_EDGE = 80
N_CHUNKS = EDGES_PER_W // K_EDGE
N_PAD = 10240
ROWS_PER_TILE = N_PAD // NS


def _sc_aggregate(table, idx_src, idx_rel, idx_dst, zeros):
  @functools.partial(
      pl.kernel,
      out_type=jax.ShapeDtypeStruct((2, N_PAD, DIM), jnp.float32),
      mesh=plsc.VectorSubcoreMesh(core_axis_name="c", subcore_axis_name="s"),
      scratch_types=[
          pltpu.VMEM((K_EDGE,), jnp.int32),
          pltpu.VMEM((K_EDGE,), jnp.int32),
          pltpu.VMEM((K_EDGE,), jnp.int32),
          pltpu.VMEM((K_EDGE, DIM), jnp.float32),
          pltpu.VMEM((K_EDGE, DIM), jnp.float32),
          pltpu.VMEM_SHARED((N_PAD, DIM), jnp.float32),
          pltpu.SemaphoreType.DMA,
      ],
  )
  def k(table_hbm, isrc_hbm, irel_hbm, idst_hbm, zeros_hbm, out_hbm,
        i1, i2, idst, rows1, rows2, acc, sem):
    cid = lax.axis_index("c")
    sid = lax.axis_index("s")
    wid = sid * NC + cid
    pltpu.sync_copy(zeros_hbm.at[pl.ds(sid * ROWS_PER_TILE, ROWS_PER_TILE)],
                    acc.at[pl.ds(sid * ROWS_PER_TILE, ROWS_PER_TILE)])
    plsc.subcore_barrier()

    base = wid * EDGES_PER_W

    def body(j, carry):
      off = base + j * K_EDGE
      pltpu.sync_copy(isrc_hbm.at[pl.ds(off, K_EDGE)], i1)
      pltpu.sync_copy(irel_hbm.at[pl.ds(off, K_EDGE)], i2)
      pltpu.sync_copy(idst_hbm.at[pl.ds(off, K_EDGE)], idst)
      cp1 = pltpu.async_copy(table_hbm.at[i1], rows1, sem)
      cp2 = pltpu.async_copy(table_hbm.at[i2], rows2, sem)
      cp1.wait()
      cp2.wait()
      pltpu.sync_copy(rows1, acc.at[idst], add=True)
      pltpu.sync_copy(rows2, acc.at[idst], add=True)
      return carry

    lax.fori_loop(0, N_CHUNKS, body, 0)
    plsc.subcore_barrier()
    pltpu.sync_copy(acc.at[pl.ds(sid * ROWS_PER_TILE, ROWS_PER_TILE)],
                    out_hbm.at[cid].at[pl.ds(sid * ROWS_PER_TILE, ROWS_PER_TILE)])

  return k(table, idx_src, idx_rel, idx_dst, zeros)


def _sc_gather(table, idx, n_rows):
  per_w = n_rows // NW
  ch = 128
  n_ch = per_w // ch

  @functools.partial(
      pl.kernel,
      out_type=jax.ShapeDtypeStruct((n_rows, DIM), jnp.float32),
      mesh=plsc.VectorSubcoreMesh(core_axis_name="c", subcore_axis_name="s"),
      scratch_types=[
          pltpu.VMEM((ch,), jnp.int32),
          pltpu.VMEM((ch, DIM), jnp.float32),
          pltpu.SemaphoreType.DMA,
      ],
  )
  def k(table_hbm, idx_hbm, out_hbm, idx_v, rows_v, sem):
    cid = lax.axis_index("c")
    sid = lax.axis_index("s")
    wid = sid * NC + cid
    base = wid * per_w

    def body(j, carry):
      off = base + j * ch
      pltpu.sync_copy(idx_hbm.at[pl.ds(off, ch)], idx_v)
      pltpu.async_copy(table_hbm.at[idx_v], rows_v, sem).wait()
      pltpu.sync_copy(rows_v, out_hbm.at[pl.ds(off, ch)])
      return carry

    lax.fori_loop(0, n_ch, body, 0)

  return k(table, idx)


def _tc_layer_dense(agg0, agg1, h, w, wl):
  blk = 1000

  def body(a0_ref, a1_ref, h_ref, w_ref, wl_ref, o_ref):
    agg = a0_ref[...] + a1_ref[...]
    o_ref[...] = jnp.tanh(
        jnp.dot(agg, w_ref[...], preferred_element_type=jnp.float32)
        + jnp.dot(h_ref[...], wl_ref[...], preferred_element_type=jnp.float32))

  return pl.pallas_call(
      body,
      grid=(N_NODES // blk,),
      in_specs=[
          pl.BlockSpec((blk, DIM), lambda i: (i, 0)),
          pl.BlockSpec((blk, DIM), lambda i: (i, 0)),
          pl.BlockSpec((blk, DIM), lambda i: (i, 0)),
          pl.BlockSpec((DIM, DIM), lambda i: (0, 0)),
          pl.BlockSpec((DIM, DIM), lambda i: (0, 0)),
      ],
      out_specs=pl.BlockSpec((blk, DIM), lambda i: (i, 0)),
      out_shape=jax.ShapeDtypeStruct((N_NODES, DIM), jnp.float32),
  )(agg0, agg1, h, w, wl)


def _tc_rel_dense(r, wr1, wr2):
  def body(r_ref, w1_ref, w2_ref, o1_ref, o2_ref):
    r1 = jnp.dot(r_ref[...], w1_ref[...], preferred_element_type=jnp.float32)
    o1_ref[...] = r1
    o2_ref[...] = jnp.dot(r1, w2_ref[...], preferred_element_type=jnp.float32)

  return pl.pallas_call(
      body,
      out_shape=[
          jax.ShapeDtypeStruct((N_REL, DIM), jnp.float32),
          jax.ShapeDtypeStruct((N_REL, DIM), jnp.float32),
      ],
  )(r, wr1, wr2)


def _tc_score(he, te, re):
  blk = 2048
  d = DIM // 2

  def body(h_ref, t_ref, r_ref, o_ref):
    hv = h_ref[...]
    tv = t_ref[...]
    rv = r_ref[...]
    hr, hi = hv[:, :d], hv[:, d:]
    tr, ti = tv[:, :d], tv[:, d:]
    rr, ri = rv[:, :d], rv[:, d:]
    o_ref[...] = jnp.sum(tr * (hr * rr - hi * ri) + ti * (hr * ri + hi * rr),
                         axis=1)

  return pl.pallas_call(
      body,
      grid=(BATCH // blk,),
      in_specs=[
          pl.BlockSpec((blk, DIM), lambda i: (i, 0)),
          pl.BlockSpec((blk, DIM), lambda i: (i, 0)),
          pl.BlockSpec((blk, DIM), lambda i: (i, 0)),
      ],
      out_specs=pl.BlockSpec((blk,), lambda i: (i,)),
      out_shape=jax.ShapeDtypeStruct((BATCH,), jnp.float32),
  )(he, te, re)


def kernel(edge_index, edge_type, head_idx, tail_idx, rel_idx,
           node_embed, rel_embed, W1, Wl1, Wr1, W2, Wl2, Wr2):
  src = edge_index[0].astype(jnp.int32)
  dst = edge_index[1].astype(jnp.int32)
  irel = edge_type.astype(jnp.int32) + N_NODES
  zeros = jnp.zeros((N_PAD, DIM), jnp.float32)

  h0 = node_embed
  r0 = rel_embed

  r1, r2 = _tc_rel_dense(r0, Wr1, Wr2)

  t1 = jnp.concatenate([h0, -r0], axis=0)
  agg1 = _sc_aggregate(t1, src, irel, dst, zeros)
  h1 = _tc_layer_dense(agg1[0, :N_NODES], agg1[1, :N_NODES], h0, W1, Wl1)

  t2 = jnp.concatenate([h1, -r1], axis=0)
  agg2 = _sc_aggregate(t2, src, irel, dst, zeros)
  h2 = _tc_layer_dense(agg2[0, :N_NODES], agg2[1, :N_NODES], h1, W2, Wl2)

  td = jnp.concatenate([h2, r2], axis=0)
  gidx = jnp.concatenate([
      head_idx.astype(jnp.int32),
      tail_idx.astype(jnp.int32),
      rel_idx.astype(jnp.int32) + N_NODES,
  ])
  rows = _sc_gather(td, gidx, 3 * BATCH)
  he = rows[:BATCH]
  te = rows[BATCH:2 * BATCH]
  re = rows[2 * BATCH:]
  return _tc_score(he, te, re)
